# Initial kernel scaffold; baseline (speedup 1.0000x reference)
import functools

import jax
import jax.numpy as jnp
from jax.experimental import pallas as pl
from jax.experimental.pallas import tpu as pltpu

_R = 512
_LOG_R = 9
_COLS = (0, 1, 3, 7, 15, 31, 50)
_KS = (32768, 16384, 8192, 4096, 2048, 1024, 1024)


def _bitonic_stage(arrs, K, j, m):
    key, idx = arrs[0], arrs[1]
    if j >= _LOG_R:
        axis, sh = 1, 1 << (j - _LOG_R)
    else:
        axis, sh = 0, 1 << j
    pos = jax.lax.broadcasted_iota(jnp.int32, key.shape, axis)
    upper = (pos & sh) != 0

    def partner(a):
        return jnp.where(upper, pltpu.roll(a, sh, axis=axis),
                         pltpu.roll(a, -sh, axis=axis))

    pk, pi = partner(key), partner(idx)
    gt = (key > pk) | ((key == pk) & (idx > pi))
    take = gt ^ upper
    if K < m:
        if K < _LOG_R:
            dpos = jax.lax.broadcasted_iota(jnp.int32, key.shape, 0)
            desc = ((dpos >> K) & 1) != 0
        else:
            dpos = jax.lax.broadcasted_iota(jnp.int32, key.shape, 1)
            desc = ((dpos >> (K - _LOG_R)) & 1) != 0
        take = take ^ desc
    out = [jnp.where(take, pk, key), jnp.where(take, pi, idx)]
    for a in arrs[2:]:
        out.append(jnp.where(take, partner(a), a))
    return out


def _sort_kernel(n_arr, m, *refs):
    arrs = [r[...] for r in refs[:n_arr]]
    for K in range(1, m + 1):
        for j in range(K - 1, -1, -1):
            arrs = _bitonic_stage(arrs, K, j, m)
    for o, a in zip(refs[n_arr:], arrs):
        o[...] = a


def _sorted_round(arrs):
    L = arrs[0].shape[1]
    m = _LOG_R + (L.bit_length() - 1)
    fn = pl.pallas_call(
        functools.partial(_sort_kernel, len(arrs), m),
        out_shape=[jax.ShapeDtypeStruct((_R, L), a.dtype) for a in arrs],
    )
    return fn(*arrs)


def kernel(learning_curves, mask):
    del mask
    lc = learning_curves[0]
    n = lc.shape[0]

    def lm(x):
        return x.reshape(-1, _R).T

    idx = jnp.arange(n, dtype=jnp.int32)
    cols = [lc[:, c] for c in _COLS]
    cur = [lm(cols[0]), lm(idx)] + [lm(c) for c in cols[1:]]
    parts = []
    for r in range(7):
        srt = _sorted_round(cur)
        kl = _KS[r] // _R
        parts.append(srt[1][:, :kl].T.reshape(-1).astype(jnp.float32))
        if r < 6:
            cur = [srt[2][:, kl:], srt[1][:, kl:]] + [a[:, kl:] for a in srt[3:]]
    return jnp.concatenate(parts)


# speedup vs baseline: 1.4901x; 1.4901x over previous
import functools

import jax
import jax.numpy as jnp
from jax.experimental import pallas as pl
from jax.experimental.pallas import tpu as pltpu

_R = 512
_LOG_R = 9
_COLS = (0, 1, 3, 7, 15, 31, 50)
_KS = (32768, 16384, 8192, 4096, 2048, 1024, 1024)


def _bitonic_stage(arrs, K, j, m):
    key, idx = arrs[0], arrs[1]
    if j >= _LOG_R:
        axis, sh = 1, 1 << (j - _LOG_R)
    else:
        axis, sh = 0, 1 << j
    pos = jax.lax.broadcasted_iota(jnp.int32, key.shape, axis)
    upper = (pos & sh) != 0

    dim = key.shape[axis]

    def partner(a):
        return jnp.where(upper, pltpu.roll(a, sh, axis=axis),
                         pltpu.roll(a, dim - sh, axis=axis))

    pk, pi = partner(key), partner(idx)
    gt = (key > pk) | ((key == pk) & (idx > pi))
    take = gt ^ upper
    if K < m:
        if K < _LOG_R:
            dpos = jax.lax.broadcasted_iota(jnp.int32, key.shape, 0)
            desc = ((dpos >> K) & 1) != 0
        else:
            dpos = jax.lax.broadcasted_iota(jnp.int32, key.shape, 1)
            desc = ((dpos >> (K - _LOG_R)) & 1) != 0
        take = take ^ desc
    out = [jnp.where(take, pk, key), jnp.where(take, pi, idx)]
    for a in arrs[2:]:
        out.append(jnp.where(take, partner(a), a))
    return out


def _sort_kernel(n_arr, m, *refs):
    arrs = [r[...] for r in refs[:n_arr]]
    for K in range(1, m + 1):
        for j in range(K - 1, -1, -1):
            arrs = _bitonic_stage(arrs, K, j, m)
    for o, a in zip(refs[n_arr:], arrs):
        o[...] = a


def _sorted_round(arrs):
    L = arrs[0].shape[1]
    m = _LOG_R + (L.bit_length() - 1)
    fn = pl.pallas_call(
        functools.partial(_sort_kernel, len(arrs), m),
        out_shape=[jax.ShapeDtypeStruct((_R, L), a.dtype) for a in arrs],
    )
    return fn(*arrs)


def kernel(learning_curves, mask):
    del mask
    lc = learning_curves[0]
    n = lc.shape[0]

    def lm(x):
        return x.reshape(-1, _R).T

    idx = jnp.arange(n, dtype=jnp.int32)
    cols = [lc[:, c] for c in _COLS]
    cur = [lm(cols[0]), lm(idx)] + [lm(c) for c in cols[1:]]
    parts = []
    for r in range(7):
        srt = _sorted_round(cur)
        kl = _KS[r] // _R
        parts.append(srt[1][:, :kl].T.reshape(-1).astype(jnp.float32))
        if r < 6:
            cur = [srt[2][:, kl:], srt[1][:, kl:]] + [a[:, kl:] for a in srt[3:]]
    return jnp.concatenate(parts)


# speedup vs baseline: 2.6569x; 1.7830x over previous
import functools

import jax
import jax.numpy as jnp
from jax.experimental import pallas as pl
from jax.experimental.pallas import tpu as pltpu

_L = 128
_COLS = (0, 1, 3, 7, 15, 31, 50)


def _bitonic_stage(arrs, K, j, m, log_r):
    key, idx = arrs[0], arrs[1]
    if j >= log_r:
        axis, sh = 1, 1 << (j - log_r)
    else:
        axis, sh = 0, 1 << j
    pos = jax.lax.broadcasted_iota(jnp.int32, key.shape, axis)
    upper = (pos & sh) != 0
    dim = key.shape[axis]

    def partner(a):
        return jnp.where(upper, pltpu.roll(a, sh, axis=axis),
                         pltpu.roll(a, dim - sh, axis=axis))

    pk, pi = partner(key), partner(idx)
    gt = (key > pk) | ((key == pk) & (idx > pi))
    take = gt ^ upper
    if K < m:
        if K < log_r:
            dpos = jax.lax.broadcasted_iota(jnp.int32, key.shape, 0)
            desc = ((dpos >> K) & 1) != 0
        else:
            dpos = jax.lax.broadcasted_iota(jnp.int32, key.shape, 1)
            desc = ((dpos >> (K - log_r)) & 1) != 0
        take = take ^ desc
    out = [jnp.where(take, pk, key), jnp.where(take, pi, idx)]
    for a in arrs[2:]:
        out.append(jnp.where(take, partner(a), a))
    return out


def _sort_kernel(n_arr, m, log_r, *refs):
    arrs = [r[...] for r in refs[:n_arr]]
    for K in range(1, m + 1):
        for j in range(K - 1, -1, -1):
            arrs = _bitonic_stage(arrs, K, j, m, log_r)
    for o, a in zip(refs[n_arr:], arrs):
        o[...] = a


def _sorted_round(arrs):
    rr = arrs[0].shape[0]
    log_r = rr.bit_length() - 1
    m = log_r + 7
    fn = pl.pallas_call(
        functools.partial(_sort_kernel, len(arrs), m, log_r),
        out_shape=[jax.ShapeDtypeStruct((rr, _L), a.dtype) for a in arrs],
    )
    return fn(*arrs)


def _survivor_half(a):
    rr = a.shape[0]
    s = a[:, 64:]
    return s.reshape(2, rr // 2, 64).transpose(1, 2, 0).reshape(rr // 2, _L)


def kernel(learning_curves, mask):
    del mask
    lc = learning_curves[0]
    n = lc.shape[0]
    rr = n // _L

    def lm(x):
        return x.reshape(_L, -1).T

    idx = jnp.arange(n, dtype=jnp.int32)
    cols = [lc[:, c] for c in _COLS]
    cur = [lm(cols[0]), lm(idx)] + [lm(c) for c in cols[1:]]
    parts = []
    for r in range(7):
        srt = _sorted_round(cur)
        if r < 6:
            parts.append(srt[1][:, :64].T.reshape(-1).astype(jnp.float32))
            cur = [_survivor_half(a) for a in [srt[2], srt[1], *srt[3:]]]
        else:
            parts.append(srt[1].T.reshape(-1).astype(jnp.float32))
    return jnp.concatenate(parts)


# speedup vs baseline: 3.5682x; 1.3430x over previous
import functools

import jax
import jax.numpy as jnp
from jax import lax
from jax.experimental import pallas as pl
from jax.experimental.pallas import tpu as pltpu
from jax.experimental.pallas import tpu_sc as plsc

_L = 128
_COLS = (0, 1, 3, 7, 15, 31, 50)
_NCURVES = 65536
_NBUD = 51


def _bitonic_stage(arrs, K, j, m, log_r):
    key, idx = arrs[0], arrs[1]
    if j >= log_r:
        axis, sh = 1, 1 << (j - log_r)
    else:
        axis, sh = 0, 1 << j
    pos = jax.lax.broadcasted_iota(jnp.int32, key.shape, axis)
    upper = (pos & sh) != 0
    dim = key.shape[axis]

    def partner(a):
        return jnp.where(upper, pltpu.roll(a, sh, axis=axis),
                         pltpu.roll(a, dim - sh, axis=axis))

    pk, pi = partner(key), partner(idx)
    gt = (key > pk) | ((key == pk) & (idx > pi))
    take = gt ^ upper
    if K < m:
        if K < log_r:
            dpos = jax.lax.broadcasted_iota(jnp.int32, key.shape, 0)
            desc = ((dpos >> K) & 1) != 0
        else:
            dpos = jax.lax.broadcasted_iota(jnp.int32, key.shape, 1)
            desc = ((dpos >> (K - log_r)) & 1) != 0
        take = take ^ desc
    out = [jnp.where(take, pk, key), jnp.where(take, pi, idx)]
    for a in arrs[2:]:
        out.append(jnp.where(take, partner(a), a))
    return out


def _sort_kernel(n_arr, m, log_r, *refs):
    arrs = [r[...] for r in refs[:n_arr]]
    for K in range(1, m + 1):
        for j in range(K - 1, -1, -1):
            arrs = _bitonic_stage(arrs, K, j, m, log_r)
    refs[n_arr][...] = arrs[1]


def _sorted_idx(arrs):
    rr = arrs[0].shape[0]
    log_r = rr.bit_length() - 1
    m = log_r + 7
    fn = pl.pallas_call(
        functools.partial(_sort_kernel, len(arrs), m, log_r),
        out_shape=jax.ShapeDtypeStruct((rr, _L), jnp.int32),
    )
    return fn(*arrs)


def _make_sc_gather(m_elems, col):
    per = m_elems // 32
    mesh = plsc.VectorSubcoreMesh(core_axis_name="c", subcore_axis_name="s")

    @functools.partial(
        pl.kernel, mesh=mesh,
        out_type=jax.ShapeDtypeStruct((m_elems,), jnp.float32),
        scratch_types=[
            pltpu.VMEM((per,), jnp.int32),
            pltpu.VMEM((per,), jnp.int32),
            pltpu.VMEM((per,), jnp.float32),
            pltpu.SemaphoreType.DMA,
        ],
    )
    def g(table_hbm, idx_hbm, out_hbm, idx_v, scaled_v, vals_v, sem):
        wid = lax.axis_index("s") * 2 + lax.axis_index("c")
        base = wid * per
        pltpu.sync_copy(idx_hbm.at[pl.ds(base, per)], idx_v)

        def body(i, carry):
            sl = pl.ds(i * 16, 16)
            scaled_v[sl] = idx_v[sl] * _NBUD + col
            return carry

        lax.fori_loop(0, per // 16, body, 0)
        pltpu.async_copy(table_hbm.at[scaled_v], vals_v, sem).wait()
        pltpu.sync_copy(vals_v, out_hbm.at[pl.ds(base, per)])

    return g


def _relayout_half(s):
    rr = s.shape[0]
    return s.reshape(2, rr // 2, 64).transpose(1, 2, 0).reshape(rr // 2, _L)


def kernel(learning_curves, mask):
    del mask
    lc = learning_curves[0]
    table = learning_curves.reshape(-1)
    n = lc.shape[0]

    def lm(x):
        return x.reshape(_L, -1).T

    idx = jnp.arange(n, dtype=jnp.int32)
    cur_key = lm(lc[:, _COLS[0]])
    cur_idx = lm(idx)
    parts = []
    for r in range(7):
        sidx = _sorted_idx([cur_key, cur_idx])
        if r < 6:
            parts.append(sidx[:, :64].T.reshape(-1).astype(jnp.float32))
            surv = _relayout_half(sidx[:, 64:])
            surv_flat = surv.T.reshape(-1)
            vals = _make_sc_gather(surv_flat.shape[0], _COLS[r + 1])(table, surv_flat)
            cur_key = vals.reshape(_L, -1).T
            cur_idx = surv
        else:
            parts.append(sidx.T.reshape(-1).astype(jnp.float32))
    return jnp.concatenate(parts)


# speedup vs baseline: 3.7833x; 1.0603x over previous
import functools

import jax
import jax.numpy as jnp
from jax import lax
from jax.experimental import pallas as pl
from jax.experimental.pallas import tpu as pltpu
from jax.experimental.pallas import tpu_sc as plsc

_L = 128
_COLS = (0, 1, 3, 7, 15, 31, 50)
_NCURVES = 65536
_NBUD = 51


def _bitonic_stage(arrs, K, j, m, log_r):
    key, idx = arrs[0], arrs[1]
    if j >= log_r:
        axis, sh = 1, 1 << (j - log_r)
    else:
        axis, sh = 0, 1 << j
    pos = jax.lax.broadcasted_iota(jnp.int32, key.shape, axis)
    upper = (pos & sh) != 0
    dim = key.shape[axis]

    def partner(a):
        return jnp.where(upper, pltpu.roll(a, sh, axis=axis),
                         pltpu.roll(a, dim - sh, axis=axis))

    pk, pi = partner(key), partner(idx)
    gt = (key > pk) | ((key == pk) & (idx > pi))
    take = gt ^ upper
    if K < m:
        if K < log_r:
            dpos = jax.lax.broadcasted_iota(jnp.int32, key.shape, 0)
            desc = ((dpos >> K) & 1) != 0
        else:
            dpos = jax.lax.broadcasted_iota(jnp.int32, key.shape, 1)
            desc = ((dpos >> (K - log_r)) & 1) != 0
        take = take ^ desc
    out = [jnp.where(take, pk, key), jnp.where(take, pi, idx)]
    for a in arrs[2:]:
        out.append(jnp.where(take, partner(a), a))
    return out


def _sort_kernel(n_arr, m, log_r, *refs):
    arrs = [r[...] for r in refs[:n_arr]]
    for K in range(1, m + 1):
        for j in range(K - 1, -1, -1):
            arrs = _bitonic_stage(arrs, K, j, m, log_r)
    refs[n_arr][...] = arrs[1]


def _sorted_idx(arrs):
    rr = arrs[0].shape[0]
    log_r = rr.bit_length() - 1
    m = log_r + 7
    fn = pl.pallas_call(
        functools.partial(_sort_kernel, len(arrs), m, log_r),
        out_shape=jax.ShapeDtypeStruct((rr, _L), jnp.int32),
    )
    return fn(*arrs)


def _make_sc_gather(m_elems, col):
    per = m_elems // 32
    mesh = plsc.VectorSubcoreMesh(core_axis_name="c", subcore_axis_name="s")

    @functools.partial(
        pl.kernel, mesh=mesh,
        out_type=jax.ShapeDtypeStruct((m_elems,), jnp.float32),
        scratch_types=[
            pltpu.VMEM((per,), jnp.int32),
            pltpu.VMEM((per,), jnp.int32),
            pltpu.VMEM((per,), jnp.float32),
            pltpu.SemaphoreType.DMA,
        ],
    )
    def g(table_hbm, idx_hbm, out_hbm, idx_v, scaled_v, vals_v, sem):
        wid = lax.axis_index("s") * 2 + lax.axis_index("c")
        base = wid * per
        pltpu.sync_copy(idx_hbm.at[pl.ds(base, per)], idx_v)

        def body(i, carry):
            sl = pl.ds(i * 16, 16)
            scaled_v[sl] = idx_v[sl] * _NBUD + col
            return carry

        lax.fori_loop(0, per // 16, body, 0)
        pltpu.async_copy(table_hbm.at[scaled_v], vals_v, sem).wait()
        pltpu.sync_copy(vals_v, out_hbm.at[pl.ds(base, per)])

    return g


def kernel(learning_curves, mask):
    del mask
    lc = learning_curves[0]
    table = learning_curves.reshape(-1)
    n = lc.shape[0]

    idx = jnp.arange(n, dtype=jnp.int32)
    cur_key = lc[:, _COLS[0]].reshape(-1, _L)
    cur_idx = idx.reshape(-1, _L)
    parts = []
    for r in range(7):
        sidx = _sorted_idx([cur_key, cur_idx])
        if r < 6:
            parts.append(sidx[:, :64].T.reshape(-1).astype(jnp.float32))
            surv_flat = sidx[:, 64:].reshape(-1)
            vals = _make_sc_gather(surv_flat.shape[0], _COLS[r + 1])(table, surv_flat)
            cur_idx = surv_flat.reshape(-1, _L)
            cur_key = vals.reshape(-1, _L)
        else:
            parts.append(sidx.T.reshape(-1).astype(jnp.float32))
    return jnp.concatenate(parts)
